# baseline (speedup 1.0000x reference)
import functools

import jax
import jax.numpy as jnp
from jax.experimental import pallas as pl
from jax.experimental.pallas import tpu as pltpu

SD = 16
HALF = SD * 2
FRAMES = HALF // 4
CONN_THRESH = 0.5
K = 100
N = 150
NP = 152
L = 256
LH = 128
KP = 104
NEG = -1e30


def _conn_half(col_ref, row_ref, arc_ref, arr_ref, h):
    lo = h * LH
    acc = jnp.zeros((NP, LH), jnp.float32)
    for f in range(FRAMES):
        b0 = 4 * f
        ax1 = col_ref[:, b0 + 0:b0 + 1]
        ay1 = col_ref[:, b0 + 1:b0 + 2]
        ax2 = col_ref[:, b0 + 2:b0 + 3]
        ay2 = col_ref[:, b0 + 3:b0 + 4]
        bx1 = row_ref[b0 + 0:b0 + 1, lo:lo + LH]
        by1 = row_ref[b0 + 1:b0 + 2, lo:lo + LH]
        bx2 = row_ref[b0 + 2:b0 + 3, lo:lo + LH]
        by2 = row_ref[b0 + 3:b0 + 4, lo:lo + LH]
        x1 = jnp.maximum(ax1, bx1)
        y1 = jnp.maximum(ay1, by1)
        x2 = jnp.minimum(ax2, bx2)
        y2 = jnp.minimum(ay2, by2)
        iw = jnp.maximum(x2 - x1 + 1.0, 0.0)
        ih = jnp.maximum(y2 - y1 + 1.0, 0.0)
        inter = iw * ih
        union = (arc_ref[:, f:f + 1] + arr_ref[f:f + 1, lo:lo + LH]) - inter
        acc = acc + inter / jnp.maximum(union, 1e-8)
    ov = acc * (1.0 / FRAMES)
    return jnp.where(ov > CONN_THRESH, ov, 0.0)


def _body(a_pl, b_tr, a_tr, b_pl, ar_a_pl, ar_b_tr, ar_a_tr, ar_b_pl,
          act_r, act_c, act0_c, out_s, out_i, s_col, s_row, *, num_g):
    g = pl.program_id(0)
    riota = jax.lax.broadcasted_iota(jnp.int32, (NP, 1), 0)
    liota = jax.lax.broadcasted_iota(jnp.int32, (1, L), 1)

    @pl.when(g == 0)
    def _init():
        s_col[:] = jnp.where(riota < N, act0_c[:], NEG)

    sc = s_col[:]
    for h in range(2):
        conn = _conn_half(a_pl, b_tr, ar_a_pl, ar_b_tr, h)
        m = jnp.max(sc + conn, axis=0, keepdims=True)
        lio = liota[:, h * LH:(h + 1) * LH]
        s_row[:, h * LH:(h + 1) * LH] = jnp.where(
            lio < N, m + act_r[:, h * LH:(h + 1) * LH], NEG)

    @pl.when(g < num_g - 1)
    def _odd():
        sr = s_row[:]
        m2 = None
        for h in range(2):
            conn2 = _conn_half(b_pl, a_tr, ar_b_pl, ar_a_tr, h)
            mh = jnp.max(sr[:, h * LH:(h + 1) * LH] + conn2,
                         axis=1, keepdims=True)
            m2 = mh if m2 is None else jnp.maximum(m2, mh)
        s_col[:] = jnp.where(riota < N, m2 + act_c[:], NEG)

    @pl.when(g == num_g - 1)
    def _topk():
        s = s_row[:]
        rr = jax.lax.broadcasted_iota(jnp.int32, (L, L), 0)
        cc = jax.lax.broadcasted_iota(jnp.int32, (L, L), 1)
        eye = (rr == cc).astype(jnp.float32)
        s_colv = jnp.sum(s * eye, axis=1, keepdims=True)
        gt = (s_colv > s).astype(jnp.int32)
        tie = ((s_colv == s) & (rr < cc)).astype(jnp.int32)
        rank = jnp.sum(gt + tie, axis=0, keepdims=True)
        k_col = jax.lax.broadcasted_iota(jnp.int32, (KP, 1), 0)
        sel = (rank == k_col).astype(jnp.float32)
        out_s[:] = jnp.sum(sel * s, axis=1, keepdims=True)
        lane_f = jax.lax.broadcasted_iota(jnp.int32, (1, L), 1).astype(jnp.float32)
        out_i[:] = jnp.sum(sel * lane_f, axis=1, keepdims=True).astype(jnp.int32)


def _areas(half_boxes):
    b = half_boxes.reshape(half_boxes.shape[0], N, FRAMES, 4)
    return (jnp.clip(b[..., 2] - b[..., 0] + 1.0, 0.0)
            * jnp.clip(b[..., 3] - b[..., 1] + 1.0, 0.0))


def kernel(p_tubes, actioness_score):
    t_clips = p_tubes.shape[0]
    num_g = t_clips // 2
    a = p_tubes[:, :, HALF:]
    b = p_tubes[:, :, :HALF]
    ar_a = _areas(a)
    ar_b = _areas(b)
    a_pl = jnp.pad(a, ((0, 0), (0, NP - N), (0, 0)))
    b_pl = jnp.pad(b, ((0, 0), (0, NP - N), (0, 0)))
    a_tr = jnp.pad(jnp.transpose(a, (0, 2, 1)), ((0, 0), (0, 0), (0, L - N)))
    b_tr = jnp.pad(jnp.transpose(b, (0, 2, 1)), ((0, 0), (0, 0), (0, L - N)))
    ar_a_pl = jnp.pad(ar_a, ((0, 0), (0, NP - N), (0, 0)))
    ar_b_pl = jnp.pad(ar_b, ((0, 0), (0, NP - N), (0, 0)))
    ar_a_tr = jnp.pad(jnp.transpose(ar_a, (0, 2, 1)), ((0, 0), (0, 0), (0, L - N)))
    ar_b_tr = jnp.pad(jnp.transpose(ar_b, (0, 2, 1)), ((0, 0), (0, 0), (0, L - N)))
    act_r = jnp.pad(actioness_score, ((0, 0), (0, L - N)))[:, None, :]
    act_c = jnp.pad(actioness_score, ((0, 0), (0, NP - N)))[:, :, None]

    last = t_clips - 1
    out_s, out_i = pl.pallas_call(
        functools.partial(_body, num_g=num_g),
        grid=(num_g,),
        in_specs=[
            pl.BlockSpec((None, NP, HALF), lambda g: (2 * g, 0, 0)),
            pl.BlockSpec((None, HALF, L), lambda g: (2 * g + 1, 0, 0)),
            pl.BlockSpec((None, HALF, L), lambda g: (2 * g + 1, 0, 0)),
            pl.BlockSpec((None, NP, HALF),
                         lambda g: (jnp.minimum(2 * g + 2, last), 0, 0)),
            pl.BlockSpec((None, NP, FRAMES), lambda g: (2 * g, 0, 0)),
            pl.BlockSpec((None, FRAMES, L), lambda g: (2 * g + 1, 0, 0)),
            pl.BlockSpec((None, FRAMES, L), lambda g: (2 * g + 1, 0, 0)),
            pl.BlockSpec((None, NP, FRAMES),
                         lambda g: (jnp.minimum(2 * g + 2, last), 0, 0)),
            pl.BlockSpec((None, 1, L), lambda g: (2 * g + 1, 0, 0)),
            pl.BlockSpec((None, NP, 1),
                         lambda g: (jnp.minimum(2 * g + 2, last), 0, 0)),
            pl.BlockSpec((None, NP, 1), lambda g: (0, 0, 0)),
        ],
        out_specs=[
            pl.BlockSpec((KP, 1), lambda g: (0, 0)),
            pl.BlockSpec((KP, 1), lambda g: (0, 0)),
        ],
        out_shape=[
            jax.ShapeDtypeStruct((KP, 1), jnp.float32),
            jax.ShapeDtypeStruct((KP, 1), jnp.int32),
        ],
        scratch_shapes=[
            pltpu.VMEM((NP, 1), jnp.float32),
            pltpu.VMEM((1, L), jnp.float32),
        ],
    )(a_pl, b_tr, a_tr, b_pl, ar_a_pl, ar_b_tr, ar_a_tr, ar_b_pl,
      act_r, act_c, act_c)
    return out_s[:K, 0], out_i[:K, 0]


# speedup vs baseline: 1.1685x; 1.1685x over previous
import functools

import jax
import jax.numpy as jnp
from jax.experimental import pallas as pl
from jax.experimental.pallas import tpu as pltpu

SD = 16
HALF = SD * 2
FRAMES = HALF // 4
CONN_THRESH = 0.5
K = 100
N = 150
NP = 152
L = 256
LH = 128
KP = 104
NEG = -1e30


def _conn(col_ref, row_ref, arc_ref, arr_ref):
    acc = jnp.zeros((NP, L), jnp.float32)
    for f in range(FRAMES):
        b0 = 4 * f
        ax1 = col_ref[:, b0 + 0:b0 + 1]
        ay1 = col_ref[:, b0 + 1:b0 + 2]
        ax2 = col_ref[:, b0 + 2:b0 + 3]
        ay2 = col_ref[:, b0 + 3:b0 + 4]
        bx1 = row_ref[b0 + 0:b0 + 1, :]
        by1 = row_ref[b0 + 1:b0 + 2, :]
        bx2 = row_ref[b0 + 2:b0 + 3, :]
        by2 = row_ref[b0 + 3:b0 + 4, :]
        x1 = jnp.maximum(ax1, bx1)
        y1 = jnp.maximum(ay1, by1)
        x2 = jnp.minimum(ax2, bx2)
        y2 = jnp.minimum(ay2, by2)
        iw = jnp.maximum(x2 - x1 + 1.0, 0.0)
        ih = jnp.maximum(y2 - y1 + 1.0, 0.0)
        inter = iw * ih
        union = (arc_ref[:, f:f + 1] + arr_ref[f:f + 1, :]) - inter
        acc = acc + inter / jnp.maximum(union, 1e-8)
    ov = acc * (1.0 / FRAMES)
    return jnp.where(ov > CONN_THRESH, ov, 0.0)


def _body(a_pl, b_tr, a_tr, b_pl, ar_a_pl, ar_b_tr, ar_a_tr, ar_b_pl,
          act_r, act_c, act0_c, out_s, out_i, s_col, s_row, *, num_g):
    g = pl.program_id(0)
    riota = jax.lax.broadcasted_iota(jnp.int32, (NP, 1), 0)
    liota = jax.lax.broadcasted_iota(jnp.int32, (1, L), 1)

    @pl.when(g == 0)
    def _init():
        s_col[:] = jnp.where(riota < N, act0_c[:], NEG)

    conn = _conn(a_pl, b_tr, ar_a_pl, ar_b_tr)
    m = jnp.max(s_col[:] + conn, axis=0, keepdims=True)
    s_row[:] = jnp.where(liota < N, m + act_r[:], NEG)

    @pl.when(g < num_g - 1)
    def _odd():
        conn2 = _conn(b_pl, a_tr, ar_b_pl, ar_a_tr)
        m2 = jnp.max(s_row[:] + conn2, axis=1, keepdims=True)
        s_col[:] = jnp.where(riota < N, m2 + act_c[:], NEG)

    @pl.when(g == num_g - 1)
    def _topk():
        s = s_row[:]
        rr = jax.lax.broadcasted_iota(jnp.int32, (L, L), 0)
        cc = jax.lax.broadcasted_iota(jnp.int32, (L, L), 1)
        eye = (rr == cc).astype(jnp.float32)
        s_colv = jnp.sum(s * eye, axis=1, keepdims=True)
        gt = (s_colv > s).astype(jnp.int32)
        tie = ((s_colv == s) & (rr < cc)).astype(jnp.int32)
        rank = jnp.sum(gt + tie, axis=0, keepdims=True)
        k_col = jax.lax.broadcasted_iota(jnp.int32, (KP, 1), 0)
        sel = (rank == k_col).astype(jnp.float32)
        out_s[:] = jnp.sum(sel * s, axis=1, keepdims=True)
        lane_f = jax.lax.broadcasted_iota(jnp.int32, (1, L), 1).astype(jnp.float32)
        out_i[:] = jnp.sum(sel * lane_f, axis=1, keepdims=True).astype(jnp.int32)


def _areas(half_boxes):
    b = half_boxes.reshape(half_boxes.shape[0], N, FRAMES, 4)
    return (jnp.clip(b[..., 2] - b[..., 0] + 1.0, 0.0)
            * jnp.clip(b[..., 3] - b[..., 1] + 1.0, 0.0))


def kernel(p_tubes, actioness_score):
    t_clips = p_tubes.shape[0]
    num_g = t_clips // 2
    a = p_tubes[:, :, HALF:]
    b = p_tubes[:, :, :HALF]
    ar_a = _areas(a)
    ar_b = _areas(b)
    a_pl = jnp.pad(a, ((0, 0), (0, NP - N), (0, 0)))
    b_pl = jnp.pad(b, ((0, 0), (0, NP - N), (0, 0)))
    a_tr = jnp.pad(jnp.transpose(a, (0, 2, 1)), ((0, 0), (0, 0), (0, L - N)))
    b_tr = jnp.pad(jnp.transpose(b, (0, 2, 1)), ((0, 0), (0, 0), (0, L - N)))
    ar_a_pl = jnp.pad(ar_a, ((0, 0), (0, NP - N), (0, 0)))
    ar_b_pl = jnp.pad(ar_b, ((0, 0), (0, NP - N), (0, 0)))
    ar_a_tr = jnp.pad(jnp.transpose(ar_a, (0, 2, 1)), ((0, 0), (0, 0), (0, L - N)))
    ar_b_tr = jnp.pad(jnp.transpose(ar_b, (0, 2, 1)), ((0, 0), (0, 0), (0, L - N)))
    act_r = jnp.pad(actioness_score, ((0, 0), (0, L - N)))[:, None, :]
    act_c = jnp.pad(actioness_score, ((0, 0), (0, NP - N)))[:, :, None]

    last = t_clips - 1
    out_s, out_i = pl.pallas_call(
        functools.partial(_body, num_g=num_g),
        grid=(num_g,),
        in_specs=[
            pl.BlockSpec((None, NP, HALF), lambda g: (2 * g, 0, 0)),
            pl.BlockSpec((None, HALF, L), lambda g: (2 * g + 1, 0, 0)),
            pl.BlockSpec((None, HALF, L), lambda g: (2 * g + 1, 0, 0)),
            pl.BlockSpec((None, NP, HALF),
                         lambda g: (jnp.minimum(2 * g + 2, last), 0, 0)),
            pl.BlockSpec((None, NP, FRAMES), lambda g: (2 * g, 0, 0)),
            pl.BlockSpec((None, FRAMES, L), lambda g: (2 * g + 1, 0, 0)),
            pl.BlockSpec((None, FRAMES, L), lambda g: (2 * g + 1, 0, 0)),
            pl.BlockSpec((None, NP, FRAMES),
                         lambda g: (jnp.minimum(2 * g + 2, last), 0, 0)),
            pl.BlockSpec((None, 1, L), lambda g: (2 * g + 1, 0, 0)),
            pl.BlockSpec((None, NP, 1),
                         lambda g: (jnp.minimum(2 * g + 2, last), 0, 0)),
            pl.BlockSpec((None, NP, 1), lambda g: (0, 0, 0)),
        ],
        out_specs=[
            pl.BlockSpec((KP, 1), lambda g: (0, 0)),
            pl.BlockSpec((KP, 1), lambda g: (0, 0)),
        ],
        out_shape=[
            jax.ShapeDtypeStruct((KP, 1), jnp.float32),
            jax.ShapeDtypeStruct((KP, 1), jnp.int32),
        ],
        scratch_shapes=[
            pltpu.VMEM((NP, 1), jnp.float32),
            pltpu.VMEM((1, L), jnp.float32),
        ],
    )(a_pl, b_tr, a_tr, b_pl, ar_a_pl, ar_b_tr, ar_a_tr, ar_b_pl,
      act_r, act_c, act_c)
    return out_s[:K, 0], out_i[:K, 0]


# speedup vs baseline: 2.1043x; 1.8009x over previous
import functools

import jax
import jax.numpy as jnp
from jax.experimental import pallas as pl
from jax.experimental.pallas import tpu as pltpu

SD = 16
HALF = SD * 2
FRAMES = HALF // 4
CONN_THRESH = 0.5
K = 100
N = 150
NP = 152
L = 256
KP = 104
NEG = -1e30


def _conn(col_ref, row_ref):
    acc = jnp.zeros((NP, L), jnp.float32)
    for f in range(FRAMES):
        b0 = 4 * f
        ax1 = col_ref[:, b0 + 0:b0 + 1]
        ay1 = col_ref[:, b0 + 1:b0 + 2]
        ax2 = col_ref[:, b0 + 2:b0 + 3]
        ay2 = col_ref[:, b0 + 3:b0 + 4]
        bx1 = row_ref[b0 + 0:b0 + 1, :]
        by1 = row_ref[b0 + 1:b0 + 2, :]
        bx2 = row_ref[b0 + 2:b0 + 3, :]
        by2 = row_ref[b0 + 3:b0 + 4, :]
        x1 = jnp.maximum(ax1, bx1)
        y1 = jnp.maximum(ay1, by1)
        x2 = jnp.minimum(ax2, bx2)
        y2 = jnp.minimum(ay2, by2)
        iw = jnp.maximum(x2 - x1 + 1.0, 0.0)
        ih = jnp.maximum(y2 - y1 + 1.0, 0.0)
        inter = iw * ih
        area_a = jnp.maximum(ax2 - ax1 + 1.0, 0.0) * jnp.maximum(ay2 - ay1 + 1.0, 0.0)
        area_b = jnp.maximum(bx2 - bx1 + 1.0, 0.0) * jnp.maximum(by2 - by1 + 1.0, 0.0)
        union = (area_a + area_b) - inter
        acc = acc + inter / jnp.maximum(union, 1e-8)
    ov = acc * (1.0 / FRAMES)
    return jnp.where(ov > CONN_THRESH, ov, 0.0)


def _body(a_pl, b_tr, act_r, act0_r, out_s, out_i, s_row, *, num_t):
    t = pl.program_id(0)
    liota = jax.lax.broadcasted_iota(jnp.int32, (1, L), 1)

    @pl.when(t == 0)
    def _init():
        s_row[:] = jnp.where(liota < N, act0_r[:], NEG)

    sc = jnp.transpose(s_row[:], (1, 0))[:NP, :]
    conn = _conn(a_pl, b_tr)
    m = jnp.max(sc + conn, axis=0, keepdims=True)
    s_row[:] = jnp.where(liota < N, m + act_r[:], NEG)

    @pl.when(t == num_t - 1)
    def _topk():
        s = s_row[:]
        rr = jax.lax.broadcasted_iota(jnp.int32, (L, L), 0)
        cc = jax.lax.broadcasted_iota(jnp.int32, (L, L), 1)
        eye = (rr == cc).astype(jnp.float32)
        s_colv = jnp.sum(s * eye, axis=1, keepdims=True)
        gt = (s_colv > s).astype(jnp.int32)
        tie = ((s_colv == s) & (rr < cc)).astype(jnp.int32)
        rank = jnp.sum(gt + tie, axis=0, keepdims=True)
        k_col = jax.lax.broadcasted_iota(jnp.int32, (KP, 1), 0)
        sel = (rank == k_col).astype(jnp.float32)
        out_s[:] = jnp.sum(sel * s, axis=1, keepdims=True)
        lane_f = jax.lax.broadcasted_iota(jnp.int32, (1, L), 1).astype(jnp.float32)
        out_i[:] = jnp.sum(sel * lane_f, axis=1, keepdims=True).astype(jnp.int32)


def kernel(p_tubes, actioness_score):
    t_clips = p_tubes.shape[0]
    num_t = t_clips - 1
    a = p_tubes[:, :, HALF:]
    b = p_tubes[:, :, :HALF]
    a_pl = jnp.pad(a, ((0, 0), (0, NP - N), (0, 0)))
    b_tr = jnp.pad(jnp.transpose(b, (0, 2, 1)), ((0, 0), (0, 0), (0, L - N)))
    act_r = jnp.pad(actioness_score, ((0, 0), (0, L - N)))[:, None, :]

    out_s, out_i = pl.pallas_call(
        functools.partial(_body, num_t=num_t),
        grid=(num_t,),
        in_specs=[
            pl.BlockSpec((None, NP, HALF), lambda t: (t, 0, 0)),
            pl.BlockSpec((None, HALF, L), lambda t: (t + 1, 0, 0)),
            pl.BlockSpec((None, 1, L), lambda t: (t + 1, 0, 0)),
            pl.BlockSpec((None, 1, L), lambda t: (0, 0, 0)),
        ],
        out_specs=[
            pl.BlockSpec((KP, 1), lambda t: (0, 0)),
            pl.BlockSpec((KP, 1), lambda t: (0, 0)),
        ],
        out_shape=[
            jax.ShapeDtypeStruct((KP, 1), jnp.float32),
            jax.ShapeDtypeStruct((KP, 1), jnp.int32),
        ],
        scratch_shapes=[
            pltpu.VMEM((1, L), jnp.float32),
        ],
    )(a_pl, b_tr, act_r, act_r)
    return out_s[:K, 0], out_i[:K, 0]
